# baseline (speedup 1.0000x reference)
import functools

import jax
import jax.numpy as jnp
from jax import lax
from jax.experimental import pallas as pl
from jax.experimental.pallas import tpu as pltpu
from jax.experimental.pallas import tpu_sc as plsc

_B = 16384
_D = 6
_NC = 2
_NS = 16
_NW = _NC * _NS
_CHUNK = 128
_G = _B // _CHUNK
_CH_PER_W = _G // _NW


def _sc_gather_soa(table_flat, idxmat):
    mesh = plsc.VectorSubcoreMesh(core_axis_name="c", subcore_axis_name="s")

    @functools.partial(
        pl.kernel,
        mesh=mesh,
        out_type=jax.ShapeDtypeStruct((_D * _G, _CHUNK), jnp.float32),
        scratch_types=[
            pltpu.VMEM((_D * _CH_PER_W, _CHUNK), jnp.int32),
            pltpu.VMEM((_D * _CH_PER_W, _CHUNK), jnp.float32),
            pltpu.SemaphoreType.DMA,
        ],
    )
    def gather_kernel(table_hbm, idx_hbm, out_hbm, idx_v, col_v, sem):
        wid = lax.axis_index("s") * _NC + lax.axis_index("c")
        for c in range(_D):
            pltpu.sync_copy(
                idx_hbm.at[pl.ds(c * _G + wid * _CH_PER_W, _CH_PER_W)],
                idx_v.at[pl.ds(c * _CH_PER_W, _CH_PER_W)])
        copies = []
        for r in range(_D * _CH_PER_W):
            copies.append(
                pltpu.async_copy(table_hbm.at[idx_v.at[r]], col_v.at[r], sem))
        for cp in copies:
            cp.wait()
        for c in range(_D):
            pltpu.sync_copy(
                col_v.at[pl.ds(c * _CH_PER_W, _CH_PER_W)],
                out_hbm.at[pl.ds(c * _G + wid * _CH_PER_W, _CH_PER_W)])

    return gather_kernel(table_flat, idxmat)


def _expmap_body(tvT_ref, out_ref):
    vx = tvT_ref[0]
    vy = tvT_ref[1]
    vz = tvT_ref[2]
    wx = tvT_ref[3]
    wy = tvT_ref[4]
    wz = tvT_ref[5]

    theta2 = wx * wx + wy * wy + wz * wz
    theta = jnp.sqrt(theta2)
    theta3 = theta2 * theta
    near = theta < 0.01
    one = jnp.ones_like(theta)
    theta_nz = jnp.where(near, one, theta)
    theta2_nz = jnp.where(near, one, theta2)
    theta3_nz = jnp.where(near, one, theta3)

    sine = jnp.sin(theta)
    cosine = jnp.where(near, 8.0 / (4.0 + theta2) - 1.0, jnp.cos(theta))
    sbt = jnp.where(near, 0.5 * cosine + 0.5, sine / theta_nz)
    omc = jnp.where(near, 0.5 * sbt, (1.0 - cosine) / theta2_nz)

    out_ref[0] = omc * wx * wx + cosine
    out_ref[1] = omc * wx * wy - sbt * wz
    out_ref[2] = omc * wx * wz + sbt * wy
    out_ref[4] = omc * wy * wx + sbt * wz
    out_ref[5] = omc * wy * wy + cosine
    out_ref[6] = omc * wy * wz - sbt * wx
    out_ref[8] = omc * wz * wx - sbt * wy
    out_ref[9] = omc * wz * wy + sbt * wx
    out_ref[10] = omc * wz * wz + cosine

    sbt2 = jnp.where(near, 1.0 - theta2 / 6.0, sbt)
    omc2 = jnp.where(near, 0.5 - theta2 / 24.0, omc)
    tms3 = jnp.where(near, 1.0 / 6.0 - theta2 / 120.0,
                     (theta - sine) / theta3_nz)
    wdotv = wx * vx + wy * vy + wz * vz
    out_ref[3] = sbt2 * vx + omc2 * (wy * vz - wz * vy) + tms3 * wdotv * wx
    out_ref[7] = sbt2 * vy + omc2 * (wz * vx - wx * vz) + tms3 * wdotv * wy
    out_ref[11] = sbt2 * vz + omc2 * (wx * vy - wy * vx) + tms3 * wdotv * wz


def _expmap(tvT3):
    return pl.pallas_call(
        _expmap_body,
        out_shape=jax.ShapeDtypeStruct((12, _G, _CHUNK), jnp.float32),
    )(tvT3)


def kernel(indices, pose_adjustment_weight):
    idx32 = indices.astype(jnp.int32)
    idxmat = ((idx32 * _D).reshape(1, _G, _CHUNK)
              + jnp.arange(_D, dtype=jnp.int32).reshape(_D, 1, 1)
              ).reshape(_D * _G, _CHUNK)
    table_flat = pose_adjustment_weight.reshape(-1)
    soa = _sc_gather_soa(table_flat, idxmat)
    out12 = _expmap(soa.reshape(_D, _G, _CHUNK))
    return out12.reshape(12, _B).T.reshape(_B, 3, 4)


# speedup vs baseline: 2.7677x; 2.7677x over previous
import functools

import jax
import jax.numpy as jnp
from jax import lax
from jax.experimental import pallas as pl
from jax.experimental.pallas import tpu as pltpu
from jax.experimental.pallas import tpu_sc as plsc

_B = 16384
_D = 6
_NC = 2
_NS = 16
_NW = _NC * _NS
_CHUNK = 128
_G = _B // _CHUNK
_CH_PER_W = _G // _NW


def _sc_gather_soa(table_flat, idxmat):
    mesh = plsc.VectorSubcoreMesh(core_axis_name="c", subcore_axis_name="s")

    @functools.partial(
        pl.kernel,
        mesh=mesh,
        out_type=jax.ShapeDtypeStruct((_D * _G, _CHUNK), jnp.float32),
        scratch_types=[
            pltpu.VMEM((_D * _CH_PER_W, _CHUNK), jnp.int32),
            pltpu.VMEM((_D * _CH_PER_W, _CHUNK), jnp.float32),
            pltpu.SemaphoreType.DMA,
        ],
    )
    def gather_kernel(table_hbm, idx_hbm, out_hbm, idx_v, col_v, sem):
        wid = lax.axis_index("s") * _NC + lax.axis_index("c")
        for c in range(_D):
            pltpu.sync_copy(
                idx_hbm.at[pl.ds(c * _G + wid * _CH_PER_W, _CH_PER_W)],
                idx_v.at[pl.ds(c * _CH_PER_W, _CH_PER_W)])
        copies = []
        for r in range(_D * _CH_PER_W):
            copies.append(
                pltpu.async_copy(table_hbm.at[idx_v.at[r]], col_v.at[r], sem))
        for cp in copies:
            cp.wait()
        for c in range(_D):
            pltpu.sync_copy(
                col_v.at[pl.ds(c * _CH_PER_W, _CH_PER_W)],
                out_hbm.at[pl.ds(c * _G + wid * _CH_PER_W, _CH_PER_W)])

    return gather_kernel(table_flat, idxmat)


def _expmap_body(tvT_ref, out_ref):
    vx = tvT_ref[0]
    vy = tvT_ref[1]
    vz = tvT_ref[2]
    wx = tvT_ref[3]
    wy = tvT_ref[4]
    wz = tvT_ref[5]

    theta2 = wx * wx + wy * wy + wz * wz
    theta = jnp.sqrt(theta2)
    theta3 = theta2 * theta
    near = theta < 0.01
    one = jnp.ones_like(theta)
    theta_nz = jnp.where(near, one, theta)
    theta2_nz = jnp.where(near, one, theta2)
    theta3_nz = jnp.where(near, one, theta3)

    sine = jnp.sin(theta)
    cosine = jnp.where(near, 8.0 / (4.0 + theta2) - 1.0, jnp.cos(theta))
    sbt = jnp.where(near, 0.5 * cosine + 0.5, sine / theta_nz)
    omc = jnp.where(near, 0.5 * sbt, (1.0 - cosine) / theta2_nz)

    out_ref[0] = omc * wx * wx + cosine
    out_ref[1] = omc * wx * wy - sbt * wz
    out_ref[2] = omc * wx * wz + sbt * wy
    out_ref[4] = omc * wy * wx + sbt * wz
    out_ref[5] = omc * wy * wy + cosine
    out_ref[6] = omc * wy * wz - sbt * wx
    out_ref[8] = omc * wz * wx - sbt * wy
    out_ref[9] = omc * wz * wy + sbt * wx
    out_ref[10] = omc * wz * wz + cosine

    sbt2 = jnp.where(near, 1.0 - theta2 / 6.0, sbt)
    omc2 = jnp.where(near, 0.5 - theta2 / 24.0, omc)
    tms3 = jnp.where(near, 1.0 / 6.0 - theta2 / 120.0,
                     (theta - sine) / theta3_nz)
    wdotv = wx * vx + wy * vy + wz * vz
    out_ref[3] = sbt2 * vx + omc2 * (wy * vz - wz * vy) + tms3 * wdotv * wx
    out_ref[7] = sbt2 * vy + omc2 * (wz * vx - wx * vz) + tms3 * wdotv * wy
    out_ref[11] = sbt2 * vz + omc2 * (wx * vy - wy * vx) + tms3 * wdotv * wz


def _expmap(tvT3):
    return pl.pallas_call(
        _expmap_body,
        out_shape=jax.ShapeDtypeStruct((12, _G, _CHUNK), jnp.float32),
    )(tvT3)


def kernel(indices, pose_adjustment_weight):
    idx32 = indices.astype(jnp.int32)
    n_cams = pose_adjustment_weight.shape[0]
    idxmat = (idx32.reshape(1, _G, _CHUNK)
              + (n_cams * jnp.arange(_D, dtype=jnp.int32)).reshape(_D, 1, 1)
              ).reshape(_D * _G, _CHUNK)
    table_flat = pose_adjustment_weight.T.reshape(-1)
    soa = _sc_gather_soa(table_flat, idxmat)
    out12 = _expmap(soa.reshape(_D, _G, _CHUNK))
    return out12.reshape(12, _B).T.reshape(_B, 3, 4)


# speedup vs baseline: 3.0149x; 1.0893x over previous
import functools

import jax
import jax.numpy as jnp
from jax import lax
from jax.experimental import pallas as pl
from jax.experimental.pallas import tpu as pltpu
from jax.experimental.pallas import tpu_sc as plsc

_B = 16384
_D = 6
_NC = 2
_NS = 16
_NW = _NC * _NS
_CHUNK = 128
_G = _B // _CHUNK
_CH_PER_W = _G // _NW


def _sc_gather_soa(table_flat, idx2d, n_cams):
    mesh = plsc.VectorSubcoreMesh(core_axis_name="c", subcore_axis_name="s")

    @functools.partial(
        pl.kernel,
        mesh=mesh,
        out_type=jax.ShapeDtypeStruct((_D * _G, _CHUNK), jnp.float32),
        scratch_types=[
            pltpu.VMEM((_CH_PER_W, _CHUNK), jnp.int32),
            pltpu.VMEM((_D * _CH_PER_W, _CHUNK), jnp.float32),
            pltpu.SemaphoreType.DMA,
            pltpu.SemaphoreType.DMA,
        ],
    )
    def gather_kernel(table_hbm, idx_hbm, out_hbm, idx_v, col_v, sem, sem_out):
        wid = lax.axis_index("s") * _NC + lax.axis_index("c")
        pltpu.sync_copy(idx_hbm.at[pl.ds(wid * _CH_PER_W, _CH_PER_W)], idx_v)
        copies = []
        for c in range(_D):
            col_ref = table_hbm.at[pl.ds(c * n_cams, n_cams)]
            for j in range(_CH_PER_W):
                copies.append(
                    pltpu.async_copy(col_ref.at[idx_v.at[j]],
                                     col_v.at[c * _CH_PER_W + j], sem))
        out_copies = []
        for c in range(_D):
            for cp in copies[c * _CH_PER_W:(c + 1) * _CH_PER_W]:
                cp.wait()
            out_copies.append(pltpu.async_copy(
                col_v.at[pl.ds(c * _CH_PER_W, _CH_PER_W)],
                out_hbm.at[pl.ds(c * _G + wid * _CH_PER_W, _CH_PER_W)],
                sem_out))
        for cp in out_copies:
            cp.wait()

    return gather_kernel(table_flat, idx2d)


def _expmap_body(tvT_ref, out_ref):
    vx = tvT_ref[0]
    vy = tvT_ref[1]
    vz = tvT_ref[2]
    wx = tvT_ref[3]
    wy = tvT_ref[4]
    wz = tvT_ref[5]

    theta2 = wx * wx + wy * wy + wz * wz
    theta = jnp.sqrt(theta2)
    theta3 = theta2 * theta
    near = theta < 0.01
    one = jnp.ones_like(theta)
    theta_nz = jnp.where(near, one, theta)
    theta2_nz = jnp.where(near, one, theta2)
    theta3_nz = jnp.where(near, one, theta3)

    sine = jnp.sin(theta)
    cosine = jnp.where(near, 8.0 / (4.0 + theta2) - 1.0, jnp.cos(theta))
    sbt = jnp.where(near, 0.5 * cosine + 0.5, sine / theta_nz)
    omc = jnp.where(near, 0.5 * sbt, (1.0 - cosine) / theta2_nz)

    out_ref[0] = omc * wx * wx + cosine
    out_ref[3] = omc * wx * wy - sbt * wz
    out_ref[6] = omc * wx * wz + sbt * wy
    out_ref[1] = omc * wy * wx + sbt * wz
    out_ref[4] = omc * wy * wy + cosine
    out_ref[7] = omc * wy * wz - sbt * wx
    out_ref[2] = omc * wz * wx - sbt * wy
    out_ref[5] = omc * wz * wy + sbt * wx
    out_ref[8] = omc * wz * wz + cosine

    sbt2 = jnp.where(near, 1.0 - theta2 / 6.0, sbt)
    omc2 = jnp.where(near, 0.5 - theta2 / 24.0, omc)
    tms3 = jnp.where(near, 1.0 / 6.0 - theta2 / 120.0,
                     (theta - sine) / theta3_nz)
    wdotv = wx * vx + wy * vy + wz * vz
    out_ref[9] = sbt2 * vx + omc2 * (wy * vz - wz * vy) + tms3 * wdotv * wx
    out_ref[10] = sbt2 * vy + omc2 * (wz * vx - wx * vz) + tms3 * wdotv * wy
    out_ref[11] = sbt2 * vz + omc2 * (wx * vy - wy * vx) + tms3 * wdotv * wz


_EG = 4


def _expmap(tvT3):
    return pl.pallas_call(
        _expmap_body,
        grid=(_EG,),
        in_specs=[pl.BlockSpec((_D, _G // _EG, _CHUNK), lambda i: (0, i, 0))],
        out_specs=pl.BlockSpec((12, _G // _EG, _CHUNK), lambda i: (0, i, 0)),
        out_shape=jax.ShapeDtypeStruct((12, _G, _CHUNK), jnp.float32),
    )(tvT3)


def kernel(indices, pose_adjustment_weight):
    n_cams = pose_adjustment_weight.shape[0]
    idx2d = indices.astype(jnp.int32).reshape(_G, _CHUNK)
    table_flat = pose_adjustment_weight.T.reshape(-1)
    soa = _sc_gather_soa(table_flat, idx2d, n_cams)
    out12 = _expmap(soa.reshape(_D, _G, _CHUNK))
    return out12.reshape(4, 3, _B).transpose(2, 1, 0)


# speedup vs baseline: 3.1297x; 1.0381x over previous
import functools

import jax
import jax.numpy as jnp
from jax import lax
from jax.experimental import pallas as pl
from jax.experimental.pallas import tpu as pltpu
from jax.experimental.pallas import tpu_sc as plsc

_B = 16384
_D = 6
_NC = 2
_NS = 16
_NW = _NC * _NS
_CHUNK = 128
_G = _B // _CHUNK
_CH_PER_W = _G // _NW


def _sc_gather_soa(table_flat, idx2d, n_cams):
    mesh = plsc.VectorSubcoreMesh(core_axis_name="c", subcore_axis_name="s")

    @functools.partial(
        pl.kernel,
        mesh=mesh,
        out_type=jax.ShapeDtypeStruct((_D * _G, _CHUNK), jnp.float32),
        scratch_types=[
            pltpu.VMEM((_CH_PER_W, _CHUNK), jnp.int32),
            pltpu.VMEM((_D * _CH_PER_W, _CHUNK), jnp.float32),
            pltpu.SemaphoreType.DMA,
            pltpu.SemaphoreType.DMA,
        ],
    )
    def gather_kernel(table_hbm, idx_hbm, out_hbm, idx_v, col_v, sem, sem_out):
        wid = lax.axis_index("s") * _NC + lax.axis_index("c")
        pltpu.sync_copy(idx_hbm.at[pl.ds(wid * _CH_PER_W, _CH_PER_W)], idx_v)
        copies = []
        for c in range(_D):
            col_ref = table_hbm.at[pl.ds(c * n_cams, n_cams)]
            for j in range(_CH_PER_W):
                copies.append(
                    pltpu.async_copy(col_ref.at[idx_v.at[j]],
                                     col_v.at[c * _CH_PER_W + j], sem))
        out_copies = []
        for c in range(_D):
            for cp in copies[c * _CH_PER_W:(c + 1) * _CH_PER_W]:
                cp.wait()
            out_copies.append(pltpu.async_copy(
                col_v.at[pl.ds(c * _CH_PER_W, _CH_PER_W)],
                out_hbm.at[pl.ds(c * _G + wid * _CH_PER_W, _CH_PER_W)],
                sem_out))
        for cp in out_copies:
            cp.wait()

    return gather_kernel(table_flat, idx2d)


def _expmap_body(tvT_ref, out_ref):
    vx = tvT_ref[0]
    vy = tvT_ref[1]
    vz = tvT_ref[2]
    wx = tvT_ref[3]
    wy = tvT_ref[4]
    wz = tvT_ref[5]

    theta2 = wx * wx + wy * wy + wz * wz
    theta = jnp.sqrt(theta2)
    theta3 = theta2 * theta
    near = theta < 0.01
    one = jnp.ones_like(theta)
    theta_nz = jnp.where(near, one, theta)
    theta2_nz = jnp.where(near, one, theta2)
    theta3_nz = jnp.where(near, one, theta3)

    sine = jnp.sin(theta)
    cosine = jnp.where(near, 8.0 / (4.0 + theta2) - 1.0, jnp.cos(theta))
    sbt = jnp.where(near, 0.5 * cosine + 0.5, sine / theta_nz)
    omc = jnp.where(near, 0.5 * sbt, (1.0 - cosine) / theta2_nz)

    out_ref[0] = omc * wx * wx + cosine
    out_ref[3] = omc * wx * wy - sbt * wz
    out_ref[6] = omc * wx * wz + sbt * wy
    out_ref[1] = omc * wy * wx + sbt * wz
    out_ref[4] = omc * wy * wy + cosine
    out_ref[7] = omc * wy * wz - sbt * wx
    out_ref[2] = omc * wz * wx - sbt * wy
    out_ref[5] = omc * wz * wy + sbt * wx
    out_ref[8] = omc * wz * wz + cosine

    sbt2 = jnp.where(near, 1.0 - theta2 / 6.0, sbt)
    omc2 = jnp.where(near, 0.5 - theta2 / 24.0, omc)
    tms3 = jnp.where(near, 1.0 / 6.0 - theta2 / 120.0,
                     (theta - sine) / theta3_nz)
    wdotv = wx * vx + wy * vy + wz * vz
    out_ref[9] = sbt2 * vx + omc2 * (wy * vz - wz * vy) + tms3 * wdotv * wx
    out_ref[10] = sbt2 * vy + omc2 * (wz * vx - wx * vz) + tms3 * wdotv * wy
    out_ref[11] = sbt2 * vz + omc2 * (wx * vy - wy * vx) + tms3 * wdotv * wz


_EG = 2


def _expmap(tvT3):
    return pl.pallas_call(
        _expmap_body,
        grid=(_EG,),
        in_specs=[pl.BlockSpec((_D, _G // _EG, _CHUNK), lambda i: (0, i, 0))],
        out_specs=pl.BlockSpec((12, _G // _EG, _CHUNK), lambda i: (0, i, 0)),
        out_shape=jax.ShapeDtypeStruct((12, _G, _CHUNK), jnp.float32),
    )(tvT3)


def kernel(indices, pose_adjustment_weight):
    n_cams = pose_adjustment_weight.shape[0]
    idx2d = indices.astype(jnp.int32).reshape(_G, _CHUNK)
    table_flat = pose_adjustment_weight.T.reshape(-1)
    soa = _sc_gather_soa(table_flat, idx2d, n_cams)
    out12 = _expmap(soa.reshape(_D, _G, _CHUNK))
    return out12.reshape(4, 3, _B).transpose(2, 1, 0)


# speedup vs baseline: 3.1332x; 1.0011x over previous
import functools

import jax
import jax.numpy as jnp
from jax import lax
from jax.experimental import pallas as pl
from jax.experimental.pallas import tpu as pltpu
from jax.experimental.pallas import tpu_sc as plsc

_B = 16384
_D = 6
_NC = 2
_NS = 16
_NW = _NC * _NS
_CHUNK = 128
_G = _B // _CHUNK
_CH_PER_W = _G // _NW


def _sc_gather_soa(table_flat, idx2d, n_cams):
    mesh = plsc.VectorSubcoreMesh(core_axis_name="c", subcore_axis_name="s")

    @functools.partial(
        pl.kernel,
        mesh=mesh,
        out_type=jax.ShapeDtypeStruct((_D * _G, _CHUNK), jnp.float32),
        scratch_types=[
            pltpu.VMEM((_CH_PER_W, _CHUNK), jnp.int32),
            pltpu.VMEM((_D * _CH_PER_W, _CHUNK), jnp.float32),
            pltpu.SemaphoreType.DMA,
            pltpu.SemaphoreType.DMA,
        ],
    )
    def gather_kernel(table_hbm, idx_hbm, out_hbm, idx_v, col_v, sem, sem_out):
        wid = lax.axis_index("s") * _NC + lax.axis_index("c")
        pltpu.sync_copy(idx_hbm.at[pl.ds(wid * _CH_PER_W, _CH_PER_W)], idx_v)
        copies = []
        for c in range(_D):
            col_ref = table_hbm.at[pl.ds(c * n_cams, n_cams)]
            for j in range(_CH_PER_W):
                copies.append(
                    pltpu.async_copy(col_ref.at[idx_v.at[j]],
                                     col_v.at[c * _CH_PER_W + j], sem))
        out_copies = []
        for c in range(_D):
            for cp in copies[c * _CH_PER_W:(c + 1) * _CH_PER_W]:
                cp.wait()
            out_copies.append(pltpu.async_copy(
                col_v.at[pl.ds(c * _CH_PER_W, _CH_PER_W)],
                out_hbm.at[pl.ds(c * _G + wid * _CH_PER_W, _CH_PER_W)],
                sem_out))
        for cp in out_copies:
            cp.wait()

    return gather_kernel(table_flat, idx2d)


def _expmap_body(tvT_ref, out_ref):
    vx = tvT_ref[0]
    vy = tvT_ref[1]
    vz = tvT_ref[2]
    wx = tvT_ref[3]
    wy = tvT_ref[4]
    wz = tvT_ref[5]

    theta2 = wx * wx + wy * wy + wz * wz
    near = theta2 < 1e-4
    one = jnp.ones_like(theta2)
    theta2_nz = jnp.where(near, one, theta2)
    inv_t = jax.lax.rsqrt(theta2_nz)
    theta = theta2_nz * inv_t
    inv_t2 = inv_t * inv_t
    inv_t3 = inv_t2 * inv_t

    sine = jnp.sin(theta)
    cosine = jnp.where(near, 8.0 / (4.0 + theta2) - 1.0, jnp.cos(theta))
    sbt = jnp.where(near, 0.5 * cosine + 0.5, sine * inv_t)
    omc = jnp.where(near, 0.5 * sbt, (1.0 - cosine) * inv_t2)

    out_ref[0] = omc * wx * wx + cosine
    out_ref[3] = omc * wx * wy - sbt * wz
    out_ref[6] = omc * wx * wz + sbt * wy
    out_ref[1] = omc * wy * wx + sbt * wz
    out_ref[4] = omc * wy * wy + cosine
    out_ref[7] = omc * wy * wz - sbt * wx
    out_ref[2] = omc * wz * wx - sbt * wy
    out_ref[5] = omc * wz * wy + sbt * wx
    out_ref[8] = omc * wz * wz + cosine

    sbt2 = jnp.where(near, 1.0 - theta2 / 6.0, sbt)
    omc2 = jnp.where(near, 0.5 - theta2 / 24.0, omc)
    tms3 = jnp.where(near, 1.0 / 6.0 - theta2 / 120.0,
                     (theta - sine) * inv_t3)
    wdotv = wx * vx + wy * vy + wz * vz
    out_ref[9] = sbt2 * vx + omc2 * (wy * vz - wz * vy) + tms3 * wdotv * wx
    out_ref[10] = sbt2 * vy + omc2 * (wz * vx - wx * vz) + tms3 * wdotv * wy
    out_ref[11] = sbt2 * vz + omc2 * (wx * vy - wy * vx) + tms3 * wdotv * wz


_EG = 2


def _expmap(tvT3):
    return pl.pallas_call(
        _expmap_body,
        grid=(_EG,),
        in_specs=[pl.BlockSpec((_D, _G // _EG, _CHUNK), lambda i: (0, i, 0))],
        out_specs=pl.BlockSpec((12, _G // _EG, _CHUNK), lambda i: (0, i, 0)),
        out_shape=jax.ShapeDtypeStruct((12, _G, _CHUNK), jnp.float32),
    )(tvT3)


def kernel(indices, pose_adjustment_weight):
    n_cams = pose_adjustment_weight.shape[0]
    idx2d = indices.astype(jnp.int32).reshape(_G, _CHUNK)
    table_flat = pose_adjustment_weight.T.reshape(-1)
    soa = _sc_gather_soa(table_flat, idx2d, n_cams)
    out12 = _expmap(soa.reshape(_D, _G, _CHUNK))
    return out12.reshape(4, 3, _B).transpose(2, 1, 0)
